# baseline (speedup 1.0000x reference)
import functools

import jax
import jax.numpy as jnp
from jax import lax
from jax.experimental import pallas as pl
from jax.experimental.pallas import tpu as pltpu
from jax.experimental.pallas import tpu_sc as plsc

B = 16384
D = 64
N = 1000000
NC = 2
NS = 16
NW = NC * NS
BPW = B // NW
CHUNK = 128
NCHUNK = BPW // CHUNK
L = 16

UB = 2048
GP = 256
NBLK = (N + UB - 1) // UB
RROWS = ((N + GP - 1) // GP) * 128

_mesh = plsc.VectorSubcoreMesh(core_axis_name="c", subcore_axis_name="s")


def _relayout_body(in_ref, out_ref):
    eye = jnp.eye(D, dtype=jnp.float32)
    t = lax.dot_general(in_ref[...], eye, (((0,), (0,)), ((), ())),
                        preferred_element_type=jnp.float32)
    for k in range(UB // GP):
        out_ref[pl.ds(k * 128, 128), pl.ds(0, 64)] = t[k * GP:k * GP + 128, :]
        out_ref[pl.ds(k * 128, 128), pl.ds(64, 64)] = (
            t[k * GP + 128:(k + 1) * GP, :])


def _relayout(embT):
    return pl.pallas_call(
        _relayout_body,
        grid=(NBLK,),
        in_specs=[pl.BlockSpec((D, UB), lambda c: (0, c))],
        out_specs=pl.BlockSpec((UB // 2, 128), lambda c: (c, 0)),
        out_shape=jax.ShapeDtypeStruct((RROWS, 128), jnp.float32),
        compiler_params=pltpu.CompilerParams(
            dimension_semantics=("parallel",)),
    )(embT)


@functools.partial(
    pl.kernel,
    mesh=_mesh,
    compiler_params=pltpu.CompilerParams(
        needs_layout_passes=False, use_tc_tiling_on_sc=True),
    out_type=jax.ShapeDtypeStruct((B,), jnp.float32),
    scratch_types=[
        pltpu.VMEM((BPW,), jnp.int32),
        pltpu.VMEM((BPW,), jnp.int32),
        pltpu.VMEM((BPW,), jnp.int32),
        pltpu.VMEM((BPW,), jnp.int32),
        pltpu.VMEM((CHUNK, 128), jnp.float32),
        pltpu.VMEM((CHUNK, 128), jnp.float32),
        pltpu.VMEM((BPW,), jnp.float32),
        pltpu.SemaphoreType.DMA,
    ],
)
def _dot_kernel(user_hbm, item_hbm, uemb_hbm, iemb_hbm,
                out_hbm, uidx, iidx, urow, irow, ubuf, ibuf, outv, sem):
    wid = lax.axis_index("s") * NC + lax.axis_index("c")
    base = wid * BPW

    pltpu.sync_copy(user_hbm.at[pl.ds(base, BPW)], uidx)
    pltpu.sync_copy(item_hbm.at[pl.ds(base, BPW)], iidx)

    def row_body(i, carry):
        sl = pl.ds(i * L, L)
        u = uidx[sl]
        v = iidx[sl]
        urow[sl] = (lax.shift_left(lax.shift_right_logical(u, 8), 7)
                    + (u & 127))
        irow[sl] = (lax.shift_left(lax.shift_right_logical(v, 8), 7)
                    + (v & 127))
        return carry

    lax.fori_loop(0, BPW // L, row_body, 0)

    iota = lax.iota(jnp.int32, L)

    for p in range(NCHUNK):
        sl = pl.ds(p * CHUNK, CHUNK)
        cu = pltpu.async_copy(uemb_hbm.at[urow.at[sl]], ubuf, sem)
        ci = pltpu.async_copy(iemb_hbm.at[irow.at[sl]], ibuf, sem)
        cu.wait()
        ci.wait()

        def pass_body(g, carry, p=p):
            dotv = jnp.zeros((L,), jnp.float32)
            r0 = p * CHUNK + g * L
            upar16 = (lax.shift_right_logical(uidx[pl.ds(r0, L)], 7) & 1) * D
            ipar16 = (lax.shift_right_logical(iidx[pl.ds(r0, L)], 7) & 1) * D
            for j in range(L):
                rl = g * L + j
                upar = upar16[j]
                ipar = ipar16[j]
                acc = (ubuf[rl, pl.ds(upar, L)]
                       * ibuf[rl, pl.ds(ipar, L)])
                for c in range(1, D // L):
                    acc = acc + (ubuf[rl, pl.ds(upar + c * L, L)]
                                 * ibuf[rl, pl.ds(ipar + c * L, L)])
                dotv = jnp.where(iota == j, jnp.sum(acc), dotv)
            outv[pl.ds(p * CHUNK + g * L, L)] = dotv
            return carry

        lax.fori_loop(0, CHUNK // L, pass_body, 0)

    pltpu.sync_copy(outv, out_hbm.at[pl.ds(base, BPW)])


@functools.partial(
    pl.kernel,
    mesh=_mesh,
    compiler_params=pltpu.CompilerParams(
        needs_layout_passes=False, use_tc_tiling_on_sc=False),
    out_type=jax.ShapeDtypeStruct((B,), jnp.float32),
    scratch_types=[
        pltpu.VMEM((BPW,), jnp.int32),
        pltpu.VMEM((BPW,), jnp.int32),
        pltpu.VMEM((BPW,), jnp.float32),
        pltpu.VMEM((BPW,), jnp.float32),
        pltpu.VMEM((BPW,), jnp.float32),
        pltpu.SemaphoreType.DMA,
    ],
)
def _bias_kernel(user_hbm, item_hbm, ub_hbm, ib_hbm,
                 out_hbm, uidx, iidx, ubv, ibv, outv, sem):
    wid = lax.axis_index("s") * NC + lax.axis_index("c")
    base = wid * BPW

    pltpu.sync_copy(user_hbm.at[pl.ds(base, BPW)], uidx)
    pltpu.sync_copy(item_hbm.at[pl.ds(base, BPW)], iidx)

    for c in range(NCHUNK):
        sl = pl.ds(c * CHUNK, CHUNK)
        pltpu.async_copy(ub_hbm.at[uidx.at[sl]], ubv.at[sl], sem)
        pltpu.async_copy(ib_hbm.at[iidx.at[sl]], ibv.at[sl], sem)
    pltpu.make_async_copy(ub_hbm.at[pl.ds(0, BPW)], ubv, sem).wait()
    pltpu.make_async_copy(ib_hbm.at[pl.ds(0, BPW)], ibv, sem).wait()

    def add_body(i, carry):
        sl = pl.ds(i * L, L)
        outv[sl] = ubv[sl] + ibv[sl]
        return carry

    lax.fori_loop(0, BPW // L, add_body, 0)

    pltpu.sync_copy(outv, out_hbm.at[pl.ds(base, BPW)])


def kernel(user, item, user_embedding, item_embedding, item_biases, user_biases):
    upk = _relayout(user_embedding.T)
    ipk = _relayout(item_embedding.T)
    ub1 = user_biases.reshape(-1)
    ib1 = item_biases.reshape(-1)
    dot = _dot_kernel(user, item, upk, ipk)
    bias = _bias_kernel(user, item, ub1, ib1)
    return dot + bias


# speedup vs baseline: 1.6477x; 1.6477x over previous
import functools

import jax
import jax.numpy as jnp
from jax import lax
from jax.experimental import pallas as pl
from jax.experimental.pallas import tpu as pltpu
from jax.experimental.pallas import tpu_sc as plsc

B = 16384
D = 64
N = 1000000
NC = 2
NS = 16
NW = NC * NS
BPW = B // NW
CHUNK = 128
NCHUNK = BPW // CHUNK
L = 16

UB = 8192
GP = 256
NBLK = (N + UB - 1) // UB
RROWS = ((N + GP - 1) // GP) * 128

_mesh = plsc.VectorSubcoreMesh(core_axis_name="c", subcore_axis_name="s")


def _relayout_body(in_ref, out_ref):
    eye = jnp.eye(D, dtype=jnp.float32)
    t = lax.dot_general(in_ref[...], eye, (((0,), (0,)), ((), ())),
                        preferred_element_type=jnp.float32)
    for k in range(UB // GP):
        out_ref[pl.ds(k * 128, 128), pl.ds(0, 64)] = t[k * GP:k * GP + 128, :]
        out_ref[pl.ds(k * 128, 128), pl.ds(64, 64)] = (
            t[k * GP + 128:(k + 1) * GP, :])


def _relayout(embT):
    return pl.pallas_call(
        _relayout_body,
        grid=(NBLK,),
        in_specs=[pl.BlockSpec((D, UB), lambda c: (0, c))],
        out_specs=pl.BlockSpec((UB // 2, 128), lambda c: (c, 0)),
        out_shape=jax.ShapeDtypeStruct((RROWS, 128), jnp.float32),
        compiler_params=pltpu.CompilerParams(
            dimension_semantics=("parallel",)),
    )(embT)


@functools.partial(
    pl.kernel,
    mesh=_mesh,
    compiler_params=pltpu.CompilerParams(
        needs_layout_passes=False, use_tc_tiling_on_sc=True),
    out_type=jax.ShapeDtypeStruct((B,), jnp.float32),
    scratch_types=[
        pltpu.VMEM((BPW,), jnp.int32),
        pltpu.VMEM((BPW,), jnp.int32),
        pltpu.VMEM((BPW,), jnp.int32),
        pltpu.VMEM((BPW,), jnp.int32),
        pltpu.VMEM((CHUNK, 128), jnp.float32),
        pltpu.VMEM((CHUNK, 128), jnp.float32),
        pltpu.VMEM((BPW,), jnp.float32),
        pltpu.SemaphoreType.DMA,
    ],
)
def _dot_kernel(user_hbm, item_hbm, uemb_hbm, iemb_hbm,
                out_hbm, uidx, iidx, urow, irow, ubuf, ibuf, outv, sem):
    wid = lax.axis_index("s") * NC + lax.axis_index("c")
    base = wid * BPW

    pltpu.sync_copy(user_hbm.at[pl.ds(base, BPW)], uidx)
    pltpu.sync_copy(item_hbm.at[pl.ds(base, BPW)], iidx)

    def row_body(i, carry):
        sl = pl.ds(i * L, L)
        u = uidx[sl]
        v = iidx[sl]
        urow[sl] = (lax.shift_left(lax.shift_right_logical(u, 8), 7)
                    + (u & 127))
        irow[sl] = (lax.shift_left(lax.shift_right_logical(v, 8), 7)
                    + (v & 127))
        return carry

    lax.fori_loop(0, BPW // L, row_body, 0)

    iota = lax.iota(jnp.int32, L)

    for p in range(NCHUNK):
        sl = pl.ds(p * CHUNK, CHUNK)
        cu = pltpu.async_copy(uemb_hbm.at[urow.at[sl]], ubuf, sem)
        ci = pltpu.async_copy(iemb_hbm.at[irow.at[sl]], ibuf, sem)
        cu.wait()
        ci.wait()

        def pass_body(g, carry, p=p):
            dotv = jnp.zeros((L,), jnp.float32)
            r0 = p * CHUNK + g * L
            upar16 = (lax.shift_right_logical(uidx[pl.ds(r0, L)], 7) & 1) * D
            ipar16 = (lax.shift_right_logical(iidx[pl.ds(r0, L)], 7) & 1) * D
            for j in range(L):
                rl = g * L + j
                upar = upar16[j]
                ipar = ipar16[j]
                acc = (ubuf[rl, pl.ds(upar, L)]
                       * ibuf[rl, pl.ds(ipar, L)])
                for c in range(1, D // L):
                    acc = acc + (ubuf[rl, pl.ds(upar + c * L, L)]
                                 * ibuf[rl, pl.ds(ipar + c * L, L)])
                dotv = jnp.where(iota == j, jnp.sum(acc), dotv)
            outv[pl.ds(p * CHUNK + g * L, L)] = dotv
            return carry

        lax.fori_loop(0, CHUNK // L, pass_body, 0)

    pltpu.sync_copy(outv, out_hbm.at[pl.ds(base, BPW)])


@functools.partial(
    pl.kernel,
    mesh=_mesh,
    compiler_params=pltpu.CompilerParams(
        needs_layout_passes=False, use_tc_tiling_on_sc=False),
    out_type=jax.ShapeDtypeStruct((B,), jnp.float32),
    scratch_types=[
        pltpu.VMEM((BPW,), jnp.int32),
        pltpu.VMEM((BPW,), jnp.int32),
        pltpu.VMEM((BPW,), jnp.float32),
        pltpu.VMEM((BPW,), jnp.float32),
        pltpu.VMEM((BPW,), jnp.float32),
        pltpu.SemaphoreType.DMA,
    ],
)
def _bias_kernel(user_hbm, item_hbm, ub_hbm, ib_hbm,
                 out_hbm, uidx, iidx, ubv, ibv, outv, sem):
    wid = lax.axis_index("s") * NC + lax.axis_index("c")
    base = wid * BPW

    pltpu.sync_copy(user_hbm.at[pl.ds(base, BPW)], uidx)
    pltpu.sync_copy(item_hbm.at[pl.ds(base, BPW)], iidx)

    for c in range(NCHUNK):
        sl = pl.ds(c * CHUNK, CHUNK)
        pltpu.async_copy(ub_hbm.at[uidx.at[sl]], ubv.at[sl], sem)
        pltpu.async_copy(ib_hbm.at[iidx.at[sl]], ibv.at[sl], sem)
    pltpu.make_async_copy(ub_hbm.at[pl.ds(0, BPW)], ubv, sem).wait()
    pltpu.make_async_copy(ib_hbm.at[pl.ds(0, BPW)], ibv, sem).wait()

    def add_body(i, carry):
        sl = pl.ds(i * L, L)
        outv[sl] = ubv[sl] + ibv[sl]
        return carry

    lax.fori_loop(0, BPW // L, add_body, 0)

    pltpu.sync_copy(outv, out_hbm.at[pl.ds(base, BPW)])


def kernel(user, item, user_embedding, item_embedding, item_biases, user_biases):
    upk = _relayout(user_embedding.T)
    ipk = _relayout(item_embedding.T)
    ub1 = user_biases.reshape(-1)
    ib1 = item_biases.reshape(-1)
    dot = _dot_kernel(user, item, upk, ipk)
    bias = _bias_kernel(user, item, ub1, ib1)
    return dot + bias


# speedup vs baseline: 1.8511x; 1.1234x over previous
import functools

import jax
import jax.numpy as jnp
from jax import lax
from jax.experimental import pallas as pl
from jax.experimental.pallas import tpu as pltpu
from jax.experimental.pallas import tpu_sc as plsc

B = 16384
D = 64
N = 1000000
NC = 2
NS = 16
NW = NC * NS
BPW = B // NW
CHUNK = 128
NCHUNK = BPW // CHUNK
L = 16

UB = 16384
GP = 256
NBLK = (N + UB - 1) // UB
RROWS = ((N + GP - 1) // GP) * 128

_mesh = plsc.VectorSubcoreMesh(core_axis_name="c", subcore_axis_name="s")


def _relayout_body(in_ref, out_ref):
    eye = jnp.eye(D, dtype=jnp.float32)
    t = lax.dot_general(in_ref[...], eye, (((0,), (0,)), ((), ())),
                        preferred_element_type=jnp.float32)
    for k in range(UB // GP):
        out_ref[pl.ds(k * 128, 128), pl.ds(0, 64)] = t[k * GP:k * GP + 128, :]
        out_ref[pl.ds(k * 128, 128), pl.ds(64, 64)] = (
            t[k * GP + 128:(k + 1) * GP, :])


def _relayout(embT):
    return pl.pallas_call(
        _relayout_body,
        grid=(NBLK,),
        in_specs=[pl.BlockSpec((D, UB), lambda c: (0, c))],
        out_specs=pl.BlockSpec((UB // 2, 128), lambda c: (c, 0)),
        out_shape=jax.ShapeDtypeStruct((RROWS, 128), jnp.float32),
        compiler_params=pltpu.CompilerParams(
            dimension_semantics=("parallel",)),
    )(embT)


@functools.partial(
    pl.kernel,
    mesh=_mesh,
    compiler_params=pltpu.CompilerParams(
        needs_layout_passes=False, use_tc_tiling_on_sc=True),
    out_type=jax.ShapeDtypeStruct((B,), jnp.float32),
    scratch_types=[
        pltpu.VMEM((BPW,), jnp.int32),
        pltpu.VMEM((BPW,), jnp.int32),
        pltpu.VMEM((BPW,), jnp.int32),
        pltpu.VMEM((BPW,), jnp.int32),
        pltpu.VMEM((CHUNK, 128), jnp.float32),
        pltpu.VMEM((CHUNK, 128), jnp.float32),
        pltpu.VMEM((BPW,), jnp.float32),
        pltpu.SemaphoreType.DMA,
    ],
)
def _dot_kernel(user_hbm, item_hbm, uemb_hbm, iemb_hbm,
                out_hbm, uidx, iidx, urow, irow, ubuf, ibuf, outv, sem):
    wid = lax.axis_index("s") * NC + lax.axis_index("c")
    base = wid * BPW

    pltpu.sync_copy(user_hbm.at[pl.ds(base, BPW)], uidx)
    pltpu.sync_copy(item_hbm.at[pl.ds(base, BPW)], iidx)

    def row_body(i, carry):
        sl = pl.ds(i * L, L)
        u = uidx[sl]
        v = iidx[sl]
        urow[sl] = (lax.shift_left(lax.shift_right_logical(u, 8), 7)
                    + (u & 127))
        irow[sl] = (lax.shift_left(lax.shift_right_logical(v, 8), 7)
                    + (v & 127))
        return carry

    lax.fori_loop(0, BPW // L, row_body, 0)

    iota = lax.iota(jnp.int32, L)

    for p in range(NCHUNK):
        sl = pl.ds(p * CHUNK, CHUNK)
        cu = pltpu.async_copy(uemb_hbm.at[urow.at[sl]], ubuf, sem)
        ci = pltpu.async_copy(iemb_hbm.at[irow.at[sl]], ibuf, sem)
        cu.wait()
        ci.wait()

        def pass_body(g, carry, p=p):
            dotv = jnp.zeros((L,), jnp.float32)
            r0 = p * CHUNK + g * L
            upar16 = (lax.shift_right_logical(uidx[pl.ds(r0, L)], 7) & 1) * D
            ipar16 = (lax.shift_right_logical(iidx[pl.ds(r0, L)], 7) & 1) * D
            for j in range(L):
                rl = g * L + j
                upar = upar16[j]
                ipar = ipar16[j]
                acc = (ubuf[rl, pl.ds(upar, L)]
                       * ibuf[rl, pl.ds(ipar, L)])
                for c in range(1, D // L):
                    acc = acc + (ubuf[rl, pl.ds(upar + c * L, L)]
                                 * ibuf[rl, pl.ds(ipar + c * L, L)])
                dotv = jnp.where(iota == j, jnp.sum(acc), dotv)
            outv[pl.ds(p * CHUNK + g * L, L)] = dotv
            return carry

        lax.fori_loop(0, CHUNK // L, pass_body, 0)

    pltpu.sync_copy(outv, out_hbm.at[pl.ds(base, BPW)])


@functools.partial(
    pl.kernel,
    mesh=_mesh,
    compiler_params=pltpu.CompilerParams(
        needs_layout_passes=False, use_tc_tiling_on_sc=False),
    out_type=jax.ShapeDtypeStruct((B,), jnp.float32),
    scratch_types=[
        pltpu.VMEM((BPW,), jnp.int32),
        pltpu.VMEM((BPW,), jnp.int32),
        pltpu.VMEM((BPW,), jnp.float32),
        pltpu.VMEM((BPW,), jnp.float32),
        pltpu.VMEM((BPW,), jnp.float32),
        pltpu.SemaphoreType.DMA,
    ],
)
def _bias_kernel(user_hbm, item_hbm, ub_hbm, ib_hbm,
                 out_hbm, uidx, iidx, ubv, ibv, outv, sem):
    wid = lax.axis_index("s") * NC + lax.axis_index("c")
    base = wid * BPW

    pltpu.sync_copy(user_hbm.at[pl.ds(base, BPW)], uidx)
    pltpu.sync_copy(item_hbm.at[pl.ds(base, BPW)], iidx)

    for c in range(NCHUNK):
        sl = pl.ds(c * CHUNK, CHUNK)
        pltpu.async_copy(ub_hbm.at[uidx.at[sl]], ubv.at[sl], sem)
        pltpu.async_copy(ib_hbm.at[iidx.at[sl]], ibv.at[sl], sem)
    pltpu.make_async_copy(ub_hbm.at[pl.ds(0, BPW)], ubv, sem).wait()
    pltpu.make_async_copy(ib_hbm.at[pl.ds(0, BPW)], ibv, sem).wait()

    def add_body(i, carry):
        sl = pl.ds(i * L, L)
        outv[sl] = ubv[sl] + ibv[sl]
        return carry

    lax.fori_loop(0, BPW // L, add_body, 0)

    pltpu.sync_copy(outv, out_hbm.at[pl.ds(base, BPW)])


def kernel(user, item, user_embedding, item_embedding, item_biases, user_biases):
    upk = _relayout(user_embedding.T)
    ipk = _relayout(item_embedding.T)
    ub1 = user_biases.reshape(-1)
    ib1 = item_biases.reshape(-1)
    dot = _dot_kernel(user, item, upk, ipk)
    bias = _bias_kernel(user, item, ub1, ib1)
    return dot + bias


# speedup vs baseline: 1.9508x; 1.0538x over previous
import functools

import jax
import jax.numpy as jnp
from jax import lax
from jax.experimental import pallas as pl
from jax.experimental.pallas import tpu as pltpu
from jax.experimental.pallas import tpu_sc as plsc

B = 16384
D = 64
N = 1000000
NC = 2
NS = 16
NW = NC * NS
BPW = B // NW
CHUNK = 128
NCHUNK = BPW // CHUNK
L = 16

UB = 32768
GP = 256
NBLK = (N + UB - 1) // UB
RROWS = ((N + GP - 1) // GP) * 128

_mesh = plsc.VectorSubcoreMesh(core_axis_name="c", subcore_axis_name="s")


def _relayout_body(in_ref, out_ref):
    eye = jnp.eye(D, dtype=jnp.float32)
    t = lax.dot_general(in_ref[...], eye, (((0,), (0,)), ((), ())),
                        preferred_element_type=jnp.float32)
    for k in range(UB // GP):
        out_ref[pl.ds(k * 128, 128), pl.ds(0, 64)] = t[k * GP:k * GP + 128, :]
        out_ref[pl.ds(k * 128, 128), pl.ds(64, 64)] = (
            t[k * GP + 128:(k + 1) * GP, :])


def _relayout(embT):
    return pl.pallas_call(
        _relayout_body,
        grid=(NBLK,),
        in_specs=[pl.BlockSpec((D, UB), lambda c: (0, c))],
        out_specs=pl.BlockSpec((UB // 2, 128), lambda c: (c, 0)),
        out_shape=jax.ShapeDtypeStruct((RROWS, 128), jnp.float32),
        compiler_params=pltpu.CompilerParams(
            dimension_semantics=("parallel",)),
    )(embT)


@functools.partial(
    pl.kernel,
    mesh=_mesh,
    compiler_params=pltpu.CompilerParams(
        needs_layout_passes=False, use_tc_tiling_on_sc=True),
    out_type=jax.ShapeDtypeStruct((B,), jnp.float32),
    scratch_types=[
        pltpu.VMEM((BPW,), jnp.int32),
        pltpu.VMEM((BPW,), jnp.int32),
        pltpu.VMEM((BPW,), jnp.int32),
        pltpu.VMEM((BPW,), jnp.int32),
        pltpu.VMEM((CHUNK, 128), jnp.float32),
        pltpu.VMEM((CHUNK, 128), jnp.float32),
        pltpu.VMEM((BPW,), jnp.float32),
        pltpu.SemaphoreType.DMA,
    ],
)
def _dot_kernel(user_hbm, item_hbm, uemb_hbm, iemb_hbm,
                out_hbm, uidx, iidx, urow, irow, ubuf, ibuf, outv, sem):
    wid = lax.axis_index("s") * NC + lax.axis_index("c")
    base = wid * BPW

    pltpu.sync_copy(user_hbm.at[pl.ds(base, BPW)], uidx)
    pltpu.sync_copy(item_hbm.at[pl.ds(base, BPW)], iidx)

    def row_body(i, carry):
        sl = pl.ds(i * L, L)
        u = uidx[sl]
        v = iidx[sl]
        urow[sl] = (lax.shift_left(lax.shift_right_logical(u, 8), 7)
                    + (u & 127))
        irow[sl] = (lax.shift_left(lax.shift_right_logical(v, 8), 7)
                    + (v & 127))
        return carry

    lax.fori_loop(0, BPW // L, row_body, 0)

    iota = lax.iota(jnp.int32, L)

    for p in range(NCHUNK):
        sl = pl.ds(p * CHUNK, CHUNK)
        cu = pltpu.async_copy(uemb_hbm.at[urow.at[sl]], ubuf, sem)
        ci = pltpu.async_copy(iemb_hbm.at[irow.at[sl]], ibuf, sem)
        cu.wait()
        ci.wait()

        def pass_body(g, carry, p=p):
            dotv = jnp.zeros((L,), jnp.float32)
            r0 = p * CHUNK + g * L
            upar16 = (lax.shift_right_logical(uidx[pl.ds(r0, L)], 7) & 1) * D
            ipar16 = (lax.shift_right_logical(iidx[pl.ds(r0, L)], 7) & 1) * D
            for j in range(L):
                rl = g * L + j
                upar = upar16[j]
                ipar = ipar16[j]
                acc = (ubuf[rl, pl.ds(upar, L)]
                       * ibuf[rl, pl.ds(ipar, L)])
                for c in range(1, D // L):
                    acc = acc + (ubuf[rl, pl.ds(upar + c * L, L)]
                                 * ibuf[rl, pl.ds(ipar + c * L, L)])
                dotv = jnp.where(iota == j, jnp.sum(acc), dotv)
            outv[pl.ds(p * CHUNK + g * L, L)] = dotv
            return carry

        lax.fori_loop(0, CHUNK // L, pass_body, 0)

    pltpu.sync_copy(outv, out_hbm.at[pl.ds(base, BPW)])


@functools.partial(
    pl.kernel,
    mesh=_mesh,
    compiler_params=pltpu.CompilerParams(
        needs_layout_passes=False, use_tc_tiling_on_sc=False),
    out_type=jax.ShapeDtypeStruct((B,), jnp.float32),
    scratch_types=[
        pltpu.VMEM((BPW,), jnp.int32),
        pltpu.VMEM((BPW,), jnp.int32),
        pltpu.VMEM((BPW,), jnp.float32),
        pltpu.VMEM((BPW,), jnp.float32),
        pltpu.VMEM((BPW,), jnp.float32),
        pltpu.SemaphoreType.DMA,
    ],
)
def _bias_kernel(user_hbm, item_hbm, ub_hbm, ib_hbm,
                 out_hbm, uidx, iidx, ubv, ibv, outv, sem):
    wid = lax.axis_index("s") * NC + lax.axis_index("c")
    base = wid * BPW

    pltpu.sync_copy(user_hbm.at[pl.ds(base, BPW)], uidx)
    pltpu.sync_copy(item_hbm.at[pl.ds(base, BPW)], iidx)

    for c in range(NCHUNK):
        sl = pl.ds(c * CHUNK, CHUNK)
        pltpu.async_copy(ub_hbm.at[uidx.at[sl]], ubv.at[sl], sem)
        pltpu.async_copy(ib_hbm.at[iidx.at[sl]], ibv.at[sl], sem)
    pltpu.make_async_copy(ub_hbm.at[pl.ds(0, BPW)], ubv, sem).wait()
    pltpu.make_async_copy(ib_hbm.at[pl.ds(0, BPW)], ibv, sem).wait()

    def add_body(i, carry):
        sl = pl.ds(i * L, L)
        outv[sl] = ubv[sl] + ibv[sl]
        return carry

    lax.fori_loop(0, BPW // L, add_body, 0)

    pltpu.sync_copy(outv, out_hbm.at[pl.ds(base, BPW)])


def kernel(user, item, user_embedding, item_embedding, item_biases, user_biases):
    upk = _relayout(user_embedding.T)
    ipk = _relayout(item_embedding.T)
    ub1 = user_biases.reshape(-1)
    ib1 = item_biases.reshape(-1)
    dot = _dot_kernel(user, item, upk, ipk)
    bias = _bias_kernel(user, item, ub1, ib1)
    return dot + bias
